# Initial kernel scaffold; baseline (speedup 1.0000x reference)
import jax
import jax.numpy as jnp
from jax.experimental import pallas as pl


def kernel(mvimages, W_patch, b_patch):
    raise NotImplementedError("write your pallas kernel here")


# speedup vs baseline: 3.8666x; 3.8666x over previous
import functools

import jax
import jax.numpy as jnp
from jax import lax
from jax.experimental import pallas as pl

N_CLUSTERS = 4
KMEANS_ITERS = 10
_HIGH = lax.Precision.HIGHEST


def _stageA_body(x_ref, o_ref):
    x = x_ref[0]
    xs = jnp.sum(x.reshape(3, 14, 16, 224), axis=1)
    xm = xs.reshape(48, 224)
    w_idx = lax.broadcasted_iota(jnp.int32, (224, 16), 0) % 16
    j_idx = lax.broadcasted_iota(jnp.int32, (224, 16), 1)
    S = (w_idx == j_idx).astype(jnp.float32)
    y = jnp.dot(xm, S, preferred_element_type=jnp.float32, precision=_HIGH)
    o_ref[0] = y * (1.0 / 196.0)


def _stageB_body(g_ref, wp_ref, bp_ref, o_ref):
    g = g_ref[0]
    wp = wp_ref[...]
    f = jnp.dot(g, wp, preferred_element_type=jnp.float32, precision=_HIGH)
    f = f + bp_ref[...]

    nrm = jnp.sqrt(jnp.sum(f * f, axis=1, keepdims=True))
    xn = f / (nrm + 1e-8)

    jj = lax.broadcasted_iota(jnp.int32, (12, N_CLUSTERS), 1)

    def assign(cent):
        cn = jnp.sqrt(jnp.sum(cent * cent, axis=1, keepdims=True))
        cu = cent / (cn + 1e-8)
        sim = lax.dot_general(
            xn, cu, (((1,), (1,)), ((), ())),
            preferred_element_type=jnp.float32, precision=_HIGH)
        m = jnp.max(sim, axis=1, keepdims=True)
        lab = jnp.min(jnp.where(sim == m, jj, N_CLUSTERS), axis=1,
                      keepdims=True)
        return (jj == lab).astype(jnp.float32)

    cent = xn[:N_CLUSTERS]
    ones12 = jnp.ones((12, 1), dtype=jnp.float32)
    for _ in range(KMEANS_ITERS):
        oh = assign(cent)
        counts = lax.dot_general(
            oh, ones12, (((0,), (0,)), ((), ())),
            preferred_element_type=jnp.float32, precision=_HIGH)
        sums = lax.dot_general(
            oh, xn, (((0,), (0,)), ((), ())),
            preferred_element_type=jnp.float32, precision=_HIGH)
        cent = jnp.where(counts > 0, sums / jnp.maximum(counts, 1.0), cent)

    oh = assign(cent)
    seg = lax.dot_general(
        oh, f, (((0,), (0,)), ((), ())),
        preferred_element_type=jnp.float32, precision=_HIGH)
    o_ref[0] = jnp.max(seg, axis=0, keepdims=True)


@jax.jit
def kernel(mvimages, W_patch, b_patch):
    B, M, C, H, W = mvimages.shape
    N = B * M
    x4 = mvimages.reshape(N, C, H, W)

    g = pl.pallas_call(
        _stageA_body,
        grid=(N,),
        in_specs=[pl.BlockSpec((1, C, H, W), lambda n: (n, 0, 0, 0))],
        out_specs=pl.BlockSpec((1, 48, 16), lambda n: (n, 0, 0)),
        out_shape=jax.ShapeDtypeStruct((N, 48, 16), jnp.float32),
    )(x4)

    g3 = g.reshape(B, M, 768)
    bp2 = b_patch.reshape(1, 768)

    out = pl.pallas_call(
        _stageB_body,
        grid=(B,),
        in_specs=[
            pl.BlockSpec((1, M, 768), lambda s: (s, 0, 0)),
            pl.BlockSpec((768, 768), lambda s: (0, 0)),
            pl.BlockSpec((1, 768), lambda s: (0, 0)),
        ],
        out_specs=pl.BlockSpec((1, 1, 768), lambda s: (s, 0, 0)),
        out_shape=jax.ShapeDtypeStruct((B, 1, 768), jnp.float32),
    )(g3, W_patch, bp2)

    return out.reshape(B, 768)


# speedup vs baseline: 11.9716x; 3.0962x over previous
import jax
import jax.numpy as jnp
from jax import lax
from jax.experimental import pallas as pl

N_CLUSTERS = 4
KMEANS_ITERS = 10
_HIGH = lax.Precision.HIGHEST
_IPB = 4


def _stageA_body(x_ref, o_ref):
    x = x_ref[...]
    xs = jnp.sum(x.reshape(_IPB, 3, 14, 16, 224), axis=2)
    xm = xs.reshape(_IPB * 48, 224)
    w_idx = lax.broadcasted_iota(jnp.int32, (224, 16), 0) % 16
    j_idx = lax.broadcasted_iota(jnp.int32, (224, 16), 1)
    S = (w_idx == j_idx).astype(jnp.float32)
    y = jnp.dot(xm, S, preferred_element_type=jnp.float32, precision=_HIGH)
    o_ref[...] = y.reshape(_IPB, 48, 16) * (1.0 / 196.0)


def _stageB_body(g_ref, wp_ref, bp_ref, o_ref):
    g = g_ref[...]
    f = jnp.dot(g, wp_ref[...], preferred_element_type=jnp.float32,
                precision=_HIGH) + bp_ref[...]

    nrm = jnp.sqrt(jnp.sum(f * f, axis=1, keepdims=True))
    xn = f / (nrm + 1e-8)

    row_s = lax.broadcasted_iota(jnp.int32, (96, 32), 0) // 12
    col_s = lax.broadcasted_iota(jnp.int32, (96, 32), 1) // 4
    col_k = lax.broadcasted_iota(jnp.int32, (96, 32), 1) % 4
    blk = row_s == col_s

    si = lax.broadcasted_iota(jnp.int32, (32, 96), 0)
    sj = lax.broadcasted_iota(jnp.int32, (32, 96), 1)
    sel = (sj == 12 * (si // 4) + si % 4).astype(jnp.float32)
    cent = jnp.dot(sel, xn, preferred_element_type=jnp.float32,
                   precision=_HIGH)

    ones96 = jnp.ones((96, 1), dtype=jnp.float32)
    neg = jnp.float32(-3e38)

    def assign(cent):
        cn = jnp.sqrt(jnp.sum(cent * cent, axis=1, keepdims=True))
        cu = cent / (cn + 1e-8)
        sim = lax.dot_general(
            xn, cu, (((1,), (1,)), ((), ())),
            preferred_element_type=jnp.float32, precision=_HIGH)
        simm = jnp.where(blk, sim, neg)
        m = jnp.max(simm, axis=1, keepdims=True)
        lab = jnp.min(jnp.where(simm == m, col_k, N_CLUSTERS), axis=1,
                      keepdims=True)
        return ((col_k == lab) & blk).astype(jnp.float32)

    for _ in range(KMEANS_ITERS):
        oh = assign(cent)
        counts = lax.dot_general(
            oh, ones96, (((0,), (0,)), ((), ())),
            preferred_element_type=jnp.float32, precision=_HIGH)
        sums = lax.dot_general(
            oh, xn, (((0,), (0,)), ((), ())),
            preferred_element_type=jnp.float32, precision=_HIGH)
        cent = jnp.where(counts > 0, sums / jnp.maximum(counts, 1.0), cent)

    oh = assign(cent)
    seg = lax.dot_general(
        oh, f, (((0,), (0,)), ((), ())),
        preferred_element_type=jnp.float32, precision=_HIGH)
    o_ref[...] = jnp.max(seg.reshape(8, 4, 768), axis=1)


@jax.jit
def kernel(mvimages, W_patch, b_patch):
    B, M, C, H, W = mvimages.shape
    N = B * M
    x4 = mvimages.reshape(N, C, H, W)

    g = pl.pallas_call(
        _stageA_body,
        grid=(N // _IPB,),
        in_specs=[pl.BlockSpec((_IPB, C, H, W), lambda n: (n, 0, 0, 0))],
        out_specs=pl.BlockSpec((_IPB, 48, 16), lambda n: (n, 0, 0)),
        out_shape=jax.ShapeDtypeStruct((N, 48, 16), jnp.float32),
    )(x4)

    g2 = g.reshape(N, 768)
    bp2 = b_patch.reshape(1, 768)

    out = pl.pallas_call(
        _stageB_body,
        in_specs=[
            pl.BlockSpec((N, 768), lambda: (0, 0)),
            pl.BlockSpec((768, 768), lambda: (0, 0)),
            pl.BlockSpec((1, 768), lambda: (0, 0)),
        ],
        out_specs=pl.BlockSpec((B, 768), lambda: (0, 0)),
        out_shape=jax.ShapeDtypeStruct((B, 768), jnp.float32),
    )(g2, W_patch, bp2)

    return out


# speedup vs baseline: 13.5775x; 1.1341x over previous
import jax
import jax.numpy as jnp
from jax import lax
from jax.experimental import pallas as pl

N_CLUSTERS = 4
KMEANS_ITERS = 10
_HIGH = lax.Precision.HIGHEST
_IPB = 8


def _stageA_body(x_ref, o_ref):
    x = x_ref[...]
    xs = jnp.sum(x.reshape(_IPB, 3, 14, 16, 224), axis=2)
    xm = xs.reshape(_IPB * 48, 224)
    w_idx = lax.broadcasted_iota(jnp.int32, (224, 16), 0) % 16
    j_idx = lax.broadcasted_iota(jnp.int32, (224, 16), 1)
    S = (w_idx == j_idx).astype(jnp.float32)
    y = jnp.dot(xm, S, preferred_element_type=jnp.float32, precision=_HIGH)
    o_ref[...] = y.reshape(_IPB, 48, 16) * (1.0 / 196.0)


def _stageB_body(g_ref, wp_ref, bp_ref, o_ref):
    g = g_ref[...]
    f = jnp.dot(g, wp_ref[...], preferred_element_type=jnp.float32,
                precision=_HIGH) + bp_ref[...]

    nrm = jnp.sqrt(jnp.sum(f * f, axis=1, keepdims=True))
    xn = f / (nrm + 1e-8)

    row_s = lax.broadcasted_iota(jnp.int32, (96, 32), 0) // 12
    col_s = lax.broadcasted_iota(jnp.int32, (96, 32), 1) // 4
    col_k = lax.broadcasted_iota(jnp.int32, (96, 32), 1) % 4
    blk = row_s == col_s

    si = lax.broadcasted_iota(jnp.int32, (32, 96), 0)
    sj = lax.broadcasted_iota(jnp.int32, (32, 96), 1)
    sel = (sj == 12 * (si // 4) + si % 4).astype(jnp.float32)
    cent = jnp.dot(sel, xn, preferred_element_type=jnp.float32,
                   precision=_HIGH)

    ones96 = jnp.ones((96, 1), dtype=jnp.float32)
    neg = jnp.float32(-3e38)

    def assign(cent):
        cn = jnp.sqrt(jnp.sum(cent * cent, axis=1, keepdims=True))
        cu = cent / (cn + 1e-8)
        sim = lax.dot_general(
            xn, cu, (((1,), (1,)), ((), ())),
            preferred_element_type=jnp.float32, precision=_HIGH)
        simm = jnp.where(blk, sim, neg)
        m = jnp.max(simm, axis=1, keepdims=True)
        lab = jnp.min(jnp.where(simm == m, col_k, N_CLUSTERS), axis=1,
                      keepdims=True)
        return ((col_k == lab) & blk).astype(jnp.float32)

    for _ in range(KMEANS_ITERS):
        oh = assign(cent)
        counts = lax.dot_general(
            oh, ones96, (((0,), (0,)), ((), ())),
            preferred_element_type=jnp.float32, precision=_HIGH)
        sums = lax.dot_general(
            oh, xn, (((0,), (0,)), ((), ())),
            preferred_element_type=jnp.float32, precision=_HIGH)
        cent = jnp.where(counts > 0, sums / jnp.maximum(counts, 1.0), cent)

    oh = assign(cent)
    seg = lax.dot_general(
        oh, f, (((0,), (0,)), ((), ())),
        preferred_element_type=jnp.float32, precision=_HIGH)
    o_ref[...] = jnp.max(seg.reshape(8, 4, 768), axis=1)


@jax.jit
def kernel(mvimages, W_patch, b_patch):
    B, M, C, H, W = mvimages.shape
    N = B * M
    x4 = mvimages.reshape(N, C, H, W)

    g = pl.pallas_call(
        _stageA_body,
        grid=(N // _IPB,),
        in_specs=[pl.BlockSpec((_IPB, C, H, W), lambda n: (n, 0, 0, 0))],
        out_specs=pl.BlockSpec((_IPB, 48, 16), lambda n: (n, 0, 0)),
        out_shape=jax.ShapeDtypeStruct((N, 48, 16), jnp.float32),
    )(x4)

    g2 = g.reshape(N, 768)
    bp2 = b_patch.reshape(1, 768)

    out = pl.pallas_call(
        _stageB_body,
        in_specs=[
            pl.BlockSpec((N, 768), lambda: (0, 0)),
            pl.BlockSpec((768, 768), lambda: (0, 0)),
            pl.BlockSpec((1, 768), lambda: (0, 0)),
        ],
        out_specs=pl.BlockSpec((B, 768), lambda: (0, 0)),
        out_shape=jax.ShapeDtypeStruct((B, 768), jnp.float32),
    )(g2, W_patch, bp2)

    return out


# speedup vs baseline: 13.6360x; 1.0043x over previous
import jax
import jax.numpy as jnp
from jax import lax
from jax.experimental import pallas as pl

N_CLUSTERS = 4
KMEANS_ITERS = 10
_HIGH = lax.Precision.HIGHEST
_IPB = 16


def _stageA_body(x_ref, o_ref):
    x = x_ref[...]
    xs = jnp.sum(x.reshape(_IPB, 3, 14, 16, 224), axis=2)
    xm = xs.reshape(_IPB * 48, 224)
    w_idx = lax.broadcasted_iota(jnp.int32, (224, 16), 0) % 16
    j_idx = lax.broadcasted_iota(jnp.int32, (224, 16), 1)
    S = (w_idx == j_idx).astype(jnp.float32)
    y = jnp.dot(xm, S, preferred_element_type=jnp.float32, precision=_HIGH)
    o_ref[...] = y.reshape(_IPB, 48, 16) * (1.0 / 196.0)


def _stageB_body(g_ref, wp_ref, bp_ref, o_ref):
    g = g_ref[...]
    f = jnp.dot(g, wp_ref[...], preferred_element_type=jnp.float32,
                precision=_HIGH) + bp_ref[...]

    nrm = jnp.sqrt(jnp.sum(f * f, axis=1, keepdims=True))
    xn = f / (nrm + 1e-8)

    row_s = lax.broadcasted_iota(jnp.int32, (96, 32), 0) // 12
    col_s = lax.broadcasted_iota(jnp.int32, (96, 32), 1) // 4
    col_k = lax.broadcasted_iota(jnp.int32, (96, 32), 1) % 4
    blk = row_s == col_s

    si = lax.broadcasted_iota(jnp.int32, (32, 96), 0)
    sj = lax.broadcasted_iota(jnp.int32, (32, 96), 1)
    sel = (sj == 12 * (si // 4) + si % 4).astype(jnp.float32)
    cent = jnp.dot(sel, xn, preferred_element_type=jnp.float32,
                   precision=_HIGH)

    ones96 = jnp.ones((96, 1), dtype=jnp.float32)
    neg = jnp.float32(-3e38)

    def assign(cent):
        cn = jnp.sqrt(jnp.sum(cent * cent, axis=1, keepdims=True))
        cu = cent / (cn + 1e-8)
        sim = lax.dot_general(
            xn, cu, (((1,), (1,)), ((), ())),
            preferred_element_type=jnp.float32, precision=_HIGH)
        simm = jnp.where(blk, sim, neg)
        m = jnp.max(simm, axis=1, keepdims=True)
        lab = jnp.min(jnp.where(simm == m, col_k, N_CLUSTERS), axis=1,
                      keepdims=True)
        return ((col_k == lab) & blk).astype(jnp.float32)

    for _ in range(KMEANS_ITERS):
        oh = assign(cent)
        counts = lax.dot_general(
            oh, ones96, (((0,), (0,)), ((), ())),
            preferred_element_type=jnp.float32, precision=_HIGH)
        sums = lax.dot_general(
            oh, xn, (((0,), (0,)), ((), ())),
            preferred_element_type=jnp.float32, precision=_HIGH)
        cent = jnp.where(counts > 0, sums / jnp.maximum(counts, 1.0), cent)

    oh = assign(cent)
    seg = lax.dot_general(
        oh, f, (((0,), (0,)), ((), ())),
        preferred_element_type=jnp.float32, precision=_HIGH)
    o_ref[...] = jnp.max(seg.reshape(8, 4, 768), axis=1)


@jax.jit
def kernel(mvimages, W_patch, b_patch):
    B, M, C, H, W = mvimages.shape
    N = B * M
    x4 = mvimages.reshape(N, C, H, W)

    g = pl.pallas_call(
        _stageA_body,
        grid=(N // _IPB,),
        in_specs=[pl.BlockSpec((_IPB, C, H, W), lambda n: (n, 0, 0, 0))],
        out_specs=pl.BlockSpec((_IPB, 48, 16), lambda n: (n, 0, 0)),
        out_shape=jax.ShapeDtypeStruct((N, 48, 16), jnp.float32),
    )(x4)

    g2 = g.reshape(N, 768)
    bp2 = b_patch.reshape(1, 768)

    out = pl.pallas_call(
        _stageB_body,
        in_specs=[
            pl.BlockSpec((N, 768), lambda: (0, 0)),
            pl.BlockSpec((768, 768), lambda: (0, 0)),
            pl.BlockSpec((1, 768), lambda: (0, 0)),
        ],
        out_specs=pl.BlockSpec((B, 768), lambda: (0, 0)),
        out_shape=jax.ShapeDtypeStruct((B, 768), jnp.float32),
    )(g2, W_patch, bp2)

    return out
